# baseline (speedup 1.0000x reference)
import functools

import jax
import jax.numpy as jnp
import numpy as np
from jax import lax
from jax.experimental import pallas as pl
from jax.experimental.pallas import tpu as pltpu
from jax.experimental.pallas import tpu_sc as plsc

B = 8
N = 4096
S = 1024
NS = 32
IN_C = 64
OUT_C = 128
RAD2 = np.float32(0.2 ** 2)

NCORES = 2
NSUB = 16
NW = NCORES * NSUB
GROUPS = B * S
G_PER_W = GROUPS // NW
NCHUNK = N // 16


def _fps_body(xs_ref, ys_ref, zs_ref, nx_ref, ny_ref, nz_ref, dist_ref):
    xs = xs_ref[...]
    ys = ys_ref[...]
    zs = zs_ref[...]
    lane = lax.broadcasted_iota(jnp.int32, (B, N), 1)
    lane128 = lax.broadcasted_iota(jnp.int32, (B, 128), 1)
    zeros128 = jnp.zeros((B, 128), jnp.float32)
    dist_ref[...] = jnp.full((B, N), 1e10, jnp.float32)

    def outer(o, far):
        def inner(j, c2):
            far, bx, by, bz = c2
            m = lane == far
            cx = jnp.sum(jnp.where(m, xs, 0.0), axis=1, keepdims=True)
            cy = jnp.sum(jnp.where(m, ys, 0.0), axis=1, keepdims=True)
            cz = jnp.sum(jnp.where(m, zs, 0.0), axis=1, keepdims=True)
            sel = lane128 == j
            bx = jnp.where(sel, cx, bx)
            by = jnp.where(sel, cy, by)
            bz = jnp.where(sel, cz, bz)
            dx = xs - cx
            dy = ys - cy
            dz = zs - cz
            d = dx * dx + dy * dy + dz * dz
            dist = jnp.minimum(dist_ref[...], d)
            dist_ref[...] = dist
            far = jnp.argmax(dist, axis=1).astype(jnp.int32).reshape(B, 1)
            return far, bx, by, bz

        far, bx, by, bz = lax.fori_loop(
            0, 128, inner, (far, zeros128, zeros128, zeros128))
        col = pl.multiple_of(o * 128, 128)
        nx_ref[:, pl.ds(col, 128)] = bx
        ny_ref[:, pl.ds(col, 128)] = by
        nz_ref[:, pl.ds(col, 128)] = bz
        return far

    lax.fori_loop(0, S // 128, outer, jnp.zeros((B, 1), jnp.int32))


def _fps(xyz):
    xyzT = jnp.transpose(xyz, (2, 0, 1))
    out = pl.pallas_call(
        _fps_body,
        out_shape=[jax.ShapeDtypeStruct((B, S), jnp.float32)] * 3,
        scratch_shapes=[pltpu.VMEM((B, N), jnp.float32)],
    )(xyzT[0], xyzT[1], xyzT[2])
    return out


def _dense_body(f_ref, c_ref, xt_ref, w1_ref, cw_ref, b1_ref, cb_ref,
                p_ref, q_ref, d_ref):
    wc = jnp.dot(w1_ref[...], cw_ref[...], preferred_element_type=jnp.float32)
    bc = jnp.dot(b1_ref[...], cw_ref[...], preferred_element_type=jnp.float32) \
        + cb_ref[...]
    feat = f_ref[0]
    p_ref[0] = jnp.dot(feat, wc, preferred_element_type=jnp.float32) + bc
    cb = c_ref[0]
    xt = xt_ref[0]
    q_ref[0] = jnp.dot(cb, wc[0:3, :], preferred_element_type=jnp.float32)
    cross = jnp.dot(cb, xt, preferred_element_type=jnp.float32)
    cc = jnp.sum(cb * cb, axis=1, keepdims=True)
    xx = jnp.sum(xt * xt, axis=0, keepdims=True)
    d_ref[0] = cc + xx - 2.0 * cross


def _dense(feat, new_xyz, xyz, W1, convW, b1, convb):
    xyzT = jnp.transpose(xyz, (0, 2, 1))
    grid = (B,)
    return pl.pallas_call(
        _dense_body,
        grid=grid,
        in_specs=[
            pl.BlockSpec((1, N, IN_C), lambda b: (b, 0, 0)),
            pl.BlockSpec((1, S, 3), lambda b: (b, 0, 0)),
            pl.BlockSpec((1, 3, N), lambda b: (b, 0, 0)),
            pl.BlockSpec((IN_C, IN_C), lambda b: (0, 0)),
            pl.BlockSpec((IN_C, OUT_C), lambda b: (0, 0)),
            pl.BlockSpec((1, IN_C), lambda b: (0, 0)),
            pl.BlockSpec((1, OUT_C), lambda b: (0, 0)),
        ],
        out_specs=[
            pl.BlockSpec((1, N, OUT_C), lambda b: (b, 0, 0)),
            pl.BlockSpec((1, S, OUT_C), lambda b: (b, 0, 0)),
            pl.BlockSpec((1, S, N), lambda b: (b, 0, 0)),
        ],
        out_shape=[
            jax.ShapeDtypeStruct((B, N, OUT_C), jnp.float32),
            jax.ShapeDtypeStruct((B, S, OUT_C), jnp.float32),
            jax.ShapeDtypeStruct((B, S, N), jnp.float32),
        ],
    )(feat, new_xyz, xyzT, W1, convW, b1.reshape(1, IN_C),
      convb.reshape(1, OUT_C))


def _sc_group_body(d_hbm, p_hbm, o_hbm,
                   dbuf, ibuf, idx32, rows, ov, sem_d, sem_g, sem_o):
    wid = lax.axis_index("s") * NCORES + lax.axis_index("c")
    g0 = wid * G_PER_W
    boff = (g0 // S) * N

    def scan(gi, par):
        iota = lax.iota(jnp.int32, 16)

        def blk(bj, cnt):
            def do():
                base = bj * 256
                pcv = jnp.zeros((16,), jnp.int32)
                for k in range(16):
                    msk = dbuf[par, pl.ds(base + k * 16, 16)] <= RAD2
                    pc = plsc.all_reduce_population_count(msk)
                    pcv = jnp.where(iota == k, pc, pcv)
                csum = plsc.cumsum(pcv)
                offv = cnt + (csum - pcv)
                for k in range(16):
                    msk = dbuf[par, pl.ds(base + k * 16, 16)] <= RAD2
                    rank = plsc.cumsum(msk.astype(jnp.int32))
                    offk = jnp.take(offv, jnp.full((16,), k, jnp.int32))
                    plsc.store_scatter(ibuf, [offk + rank - 1],
                                       iota + (base + k * 16), mask=msk)
                return cnt + csum[15]

            return lax.cond(cnt < NS, do, lambda: cnt)

        cnt = jnp.minimum(lax.fori_loop(0, 16, blk, jnp.int32(0)), NS)
        idx0 = ibuf[pl.ds(0, 16)][0]
        for r in range(2):
            v = ibuf[pl.ds(r * 16, 16)]
            pos = lax.iota(jnp.int32, 16) + r * 16
            idx32[par, pl.ds(r * 16, 16)] = jnp.where(pos < cnt, v, idx0) \
                + boff

    def reduce_ship(par, g):
        def red(r, carry):
            acc = []
            for c in range(8):
                v = rows[par, 2 * r, pl.ds(c * 16, 16)]
                w = rows[par, 2 * r + 1, pl.ds(c * 16, 16)]
                s1, s2, mx = carry[c]
                acc.append((s1 + (v + w), s2 + (v * v + w * w),
                            jnp.maximum(mx, jnp.maximum(v, w))))
            return tuple(acc)

        zero = jnp.zeros((16,), jnp.float32)
        ninf = jnp.full((16,), -jnp.inf, jnp.float32)
        carry = tuple((zero, zero, ninf) for _ in range(8))
        carry = lax.fori_loop(0, NS // 2, red, carry)

        @pl.when(g >= g0 + 2)
        def _():
            pltpu.make_async_copy(ov.at[par], o_hbm.at[g - 2],
                                  sem_o.at[par]).wait()

        for c in range(8):
            s1, s2, mx = carry[c]
            ov[par, pl.ds(c * 16, 16)] = s1
            ov[par, pl.ds(OUT_C + c * 16, 16)] = s2
            ov[par, pl.ds(2 * OUT_C + c * 16, 16)] = mx
        pltpu.async_copy(ov.at[par], o_hbm.at[g], sem_o.at[par])

    pltpu.async_copy(d_hbm.at[g0], dbuf.at[0], sem_d.at[0])

    def per_group(gi, _):
        g = g0 + gi
        par = jnp.bitwise_and(gi, 1)
        pltpu.make_async_copy(d_hbm.at[g], dbuf.at[par], sem_d.at[par]).wait()

        @pl.when(gi + 1 < G_PER_W)
        def _():
            pltpu.async_copy(d_hbm.at[g + 1], dbuf.at[1 - par],
                             sem_d.at[1 - par])

        scan(gi, par)

        @pl.when(gi > 0)
        def _():
            pp = 1 - par
            pltpu.make_async_copy(p_hbm.at[idx32.at[pp]], rows.at[pp],
                                  sem_g.at[pp]).wait()
            reduce_ship(pp, g - 1)

        pltpu.async_copy(p_hbm.at[idx32.at[par]], rows.at[par],
                         sem_g.at[par])
        return 0

    lax.fori_loop(0, G_PER_W, per_group, 0)

    lpar = (G_PER_W - 1) & 1
    pltpu.make_async_copy(p_hbm.at[idx32.at[lpar]], rows.at[lpar],
                          sem_g.at[lpar]).wait()
    reduce_ship(lpar, g0 + G_PER_W - 1)
    pltpu.make_async_copy(ov.at[1 - lpar], o_hbm.at[g0 + G_PER_W - 2],
                          sem_o.at[1 - lpar]).wait()
    pltpu.make_async_copy(ov.at[lpar], o_hbm.at[g0 + G_PER_W - 1],
                          sem_o.at[lpar]).wait()


def _sc_groups(d, p):
    mesh = plsc.VectorSubcoreMesh(core_axis_name="c", subcore_axis_name="s")
    f = pl.kernel(
        _sc_group_body,
        out_type=jax.ShapeDtypeStruct((GROUPS, 3 * OUT_C), jnp.float32),
        mesh=mesh,
        compiler_params=pltpu.CompilerParams(needs_layout_passes=False),
        scratch_types=[
            pltpu.VMEM((2, N), jnp.float32),
            pltpu.VMEM((320,), jnp.int32),
            pltpu.VMEM((2, NS), jnp.int32),
            pltpu.VMEM((2, NS, OUT_C), jnp.float32),
            pltpu.VMEM((2, 3 * OUT_C), jnp.float32),
            pltpu.SemaphoreType.DMA((2,)),
            pltpu.SemaphoreType.DMA((2,)),
            pltpu.SemaphoreType.DMA((2,)),
        ],
    )
    return f(d, p)


def _final_body(p_ref, q_ref, g_ref, b_ref, o_ref):
    s1 = p_ref[:, 0:OUT_C]
    s2 = p_ref[:, OUT_C:2 * OUT_C]
    mx = p_ref[:, 2 * OUT_C:3 * OUT_C]
    q = q_ref[...]
    cnt = jnp.float32(GROUPS * NS)
    sum_y = jnp.sum(s1, axis=0, keepdims=True) \
        - NS * jnp.sum(q, axis=0, keepdims=True)
    sum_y2 = jnp.sum(s2 - 2.0 * q * s1 + NS * q * q, axis=0, keepdims=True)
    mean = sum_y / cnt
    var = sum_y2 / cnt - mean * mean
    scale = g_ref[...] * lax.rsqrt(var + 1e-5)
    o_ref[...] = jnp.maximum((mx - q - mean) * scale + b_ref[...], 0.0)


def _finalize(packed, q, gamma, beta):
    return pl.pallas_call(
        _final_body,
        out_shape=jax.ShapeDtypeStruct((GROUPS, OUT_C), jnp.float32),
    )(packed, q, gamma.reshape(1, OUT_C), beta.reshape(1, OUT_C))


def kernel(xyz, t, points, W1, b1, convW, convb, gamma, beta):
    del t
    nx, ny, nz = _fps(xyz)
    new_xyz = jnp.stack([nx, ny, nz], axis=-1)
    feat = jnp.concatenate([xyz, points], axis=-1)
    p, q, d = _dense(feat, new_xyz, xyz, W1, convW, b1, convb)
    packed = _sc_groups(d.reshape(GROUPS, N), p.reshape(B * N, OUT_C))
    out = _finalize(packed, q.reshape(GROUPS, OUT_C), gamma, beta)
    return new_xyz, out.reshape(B, S, OUT_C)


# speedup vs baseline: 1.1111x; 1.1111x over previous
import functools

import jax
import jax.numpy as jnp
import numpy as np
from jax import lax
from jax.experimental import pallas as pl
from jax.experimental.pallas import tpu as pltpu
from jax.experimental.pallas import tpu_sc as plsc

B = 8
N = 4096
S = 1024
NS = 32
IN_C = 64
OUT_C = 128
RAD2 = np.float32(0.2 ** 2)

NCORES = 2
NSUB = 16
NW = NCORES * NSUB
GROUPS = B * S
G_PER_W = GROUPS // NW
NCHUNK = N // 16


def _fps_body(xs_ref, ys_ref, zs_ref, nx_ref, ny_ref, nz_ref, dist_ref):
    xs = xs_ref[...]
    ys = ys_ref[...]
    zs = zs_ref[...]
    lane = lax.broadcasted_iota(jnp.int32, (B, N), 1)
    lane128 = lax.broadcasted_iota(jnp.int32, (B, 128), 1)
    zeros128 = jnp.zeros((B, 128), jnp.float32)
    dist_ref[...] = jnp.full((B, N), 1e10, jnp.float32)

    def outer(o, far):
        def inner(j, c2):
            far, bx, by, bz = c2
            m = lane == far
            cx = jnp.sum(jnp.where(m, xs, 0.0), axis=1, keepdims=True)
            cy = jnp.sum(jnp.where(m, ys, 0.0), axis=1, keepdims=True)
            cz = jnp.sum(jnp.where(m, zs, 0.0), axis=1, keepdims=True)
            sel = lane128 == j
            bx = jnp.where(sel, cx, bx)
            by = jnp.where(sel, cy, by)
            bz = jnp.where(sel, cz, bz)
            dx = xs - cx
            dy = ys - cy
            dz = zs - cz
            d = dx * dx + dy * dy + dz * dz
            dist = jnp.minimum(dist_ref[...], d)
            dist_ref[...] = dist
            far = jnp.argmax(dist, axis=1).astype(jnp.int32).reshape(B, 1)
            return far, bx, by, bz

        far, bx, by, bz = lax.fori_loop(
            0, 128, inner, (far, zeros128, zeros128, zeros128))
        col = pl.multiple_of(o * 128, 128)
        nx_ref[:, pl.ds(col, 128)] = bx
        ny_ref[:, pl.ds(col, 128)] = by
        nz_ref[:, pl.ds(col, 128)] = bz
        return far

    lax.fori_loop(0, S // 128, outer, jnp.zeros((B, 1), jnp.int32))


def _fps(xyz):
    xyzT = jnp.transpose(xyz, (2, 0, 1))
    out = pl.pallas_call(
        _fps_body,
        out_shape=[jax.ShapeDtypeStruct((B, S), jnp.float32)] * 3,
        scratch_shapes=[pltpu.VMEM((B, N), jnp.float32)],
    )(xyzT[0], xyzT[1], xyzT[2])
    return out


def _dense_body(f_ref, c_ref, xt_ref, w1_ref, cw_ref, b1_ref, cb_ref,
                p_ref, q_ref, d_ref):
    wc = jnp.dot(w1_ref[...], cw_ref[...], preferred_element_type=jnp.float32)
    bc = jnp.dot(b1_ref[...], cw_ref[...], preferred_element_type=jnp.float32) \
        + cb_ref[...]
    feat = f_ref[0]
    p_ref[0] = jnp.dot(feat, wc, preferred_element_type=jnp.float32) + bc
    cb = c_ref[0]
    xt = xt_ref[0]
    q_ref[0] = jnp.dot(cb, wc[0:3, :], preferred_element_type=jnp.float32)
    cross = jnp.dot(cb, xt, preferred_element_type=jnp.float32)
    cc = jnp.sum(cb * cb, axis=1, keepdims=True)
    xx = jnp.sum(xt * xt, axis=0, keepdims=True)
    d_ref[0] = cc + xx - 2.0 * cross


def _dense(feat, new_xyz, xyz, W1, convW, b1, convb):
    xyzT = jnp.transpose(xyz, (0, 2, 1))
    grid = (B,)
    return pl.pallas_call(
        _dense_body,
        grid=grid,
        in_specs=[
            pl.BlockSpec((1, N, IN_C), lambda b: (b, 0, 0)),
            pl.BlockSpec((1, S, 3), lambda b: (b, 0, 0)),
            pl.BlockSpec((1, 3, N), lambda b: (b, 0, 0)),
            pl.BlockSpec((IN_C, IN_C), lambda b: (0, 0)),
            pl.BlockSpec((IN_C, OUT_C), lambda b: (0, 0)),
            pl.BlockSpec((1, IN_C), lambda b: (0, 0)),
            pl.BlockSpec((1, OUT_C), lambda b: (0, 0)),
        ],
        out_specs=[
            pl.BlockSpec((1, N, OUT_C), lambda b: (b, 0, 0)),
            pl.BlockSpec((1, S, OUT_C), lambda b: (b, 0, 0)),
            pl.BlockSpec((1, S, N), lambda b: (b, 0, 0)),
        ],
        out_shape=[
            jax.ShapeDtypeStruct((B, N, OUT_C), jnp.float32),
            jax.ShapeDtypeStruct((B, S, OUT_C), jnp.float32),
            jax.ShapeDtypeStruct((B, S, N), jnp.float32),
        ],
    )(feat, new_xyz, xyzT, W1, convW, b1.reshape(1, IN_C),
      convb.reshape(1, OUT_C))


def _sc_group_body(d_hbm, p_hbm, o_hbm,
                   dbuf, ibuf, idx32, rows, ov, sem_d, sem_g, sem_o):
    wid = lax.axis_index("s") * NCORES + lax.axis_index("c")
    g0 = wid * G_PER_W
    boff = (g0 // S) * N

    def scan(gi, par):
        def blk(bj, cnt):
            def do():
                c = cnt
                for k in range(16):
                    off = bj * 256 + k * 16
                    msk = dbuf[par, pl.ds(off, 16)] <= RAD2
                    idxv = lax.iota(jnp.int32, 16) + off
                    plsc.store_compressed(ibuf.at[pl.ds(c, 16)], idxv,
                                          mask=msk)
                    c = c + plsc.all_reduce_population_count(msk)[0]
                return c

            return lax.cond(cnt < NS, do, lambda: cnt)

        cnt = jnp.minimum(lax.fori_loop(0, 16, blk, jnp.int32(0)), NS)
        idx0 = ibuf[pl.ds(0, 16)][0]
        for r in range(2):
            v = ibuf[pl.ds(r * 16, 16)]
            pos = lax.iota(jnp.int32, 16) + r * 16
            idx32[par, pl.ds(r * 16, 16)] = jnp.where(pos < cnt, v, idx0) \
                + boff

    def reduce_ship(par, g):
        def red(r, carry):
            acc = []
            for c in range(8):
                v = rows[par, 2 * r, pl.ds(c * 16, 16)]
                w = rows[par, 2 * r + 1, pl.ds(c * 16, 16)]
                s1, s2, mx = carry[c]
                acc.append((s1 + (v + w), s2 + (v * v + w * w),
                            jnp.maximum(mx, jnp.maximum(v, w))))
            return tuple(acc)

        zero = jnp.zeros((16,), jnp.float32)
        ninf = jnp.full((16,), -jnp.inf, jnp.float32)
        carry = tuple((zero, zero, ninf) for _ in range(8))
        carry = lax.fori_loop(0, NS // 2, red, carry)

        @pl.when(g >= g0 + 2)
        def _():
            pltpu.make_async_copy(ov.at[par], o_hbm.at[g - 2],
                                  sem_o.at[par]).wait()

        for c in range(8):
            s1, s2, mx = carry[c]
            ov[par, pl.ds(c * 16, 16)] = s1
            ov[par, pl.ds(OUT_C + c * 16, 16)] = s2
            ov[par, pl.ds(2 * OUT_C + c * 16, 16)] = mx
        pltpu.async_copy(ov.at[par], o_hbm.at[g], sem_o.at[par])

    pltpu.async_copy(d_hbm.at[g0], dbuf.at[0], sem_d.at[0])

    def per_group(gi, _):
        g = g0 + gi
        par = jnp.bitwise_and(gi, 1)
        pltpu.make_async_copy(d_hbm.at[g], dbuf.at[par], sem_d.at[par]).wait()

        @pl.when(gi + 1 < G_PER_W)
        def _():
            pltpu.async_copy(d_hbm.at[g + 1], dbuf.at[1 - par],
                             sem_d.at[1 - par])

        scan(gi, par)

        @pl.when(gi > 0)
        def _():
            pp = 1 - par
            pltpu.make_async_copy(p_hbm.at[idx32.at[pp]], rows.at[pp],
                                  sem_g.at[pp]).wait()
            reduce_ship(pp, g - 1)

        pltpu.async_copy(p_hbm.at[idx32.at[par]], rows.at[par],
                         sem_g.at[par])
        return 0

    lax.fori_loop(0, G_PER_W, per_group, 0)

    lpar = (G_PER_W - 1) & 1
    pltpu.make_async_copy(p_hbm.at[idx32.at[lpar]], rows.at[lpar],
                          sem_g.at[lpar]).wait()
    reduce_ship(lpar, g0 + G_PER_W - 1)
    pltpu.make_async_copy(ov.at[1 - lpar], o_hbm.at[g0 + G_PER_W - 2],
                          sem_o.at[1 - lpar]).wait()
    pltpu.make_async_copy(ov.at[lpar], o_hbm.at[g0 + G_PER_W - 1],
                          sem_o.at[lpar]).wait()


def _sc_groups(d, p):
    mesh = plsc.VectorSubcoreMesh(core_axis_name="c", subcore_axis_name="s")
    f = pl.kernel(
        _sc_group_body,
        out_type=jax.ShapeDtypeStruct((GROUPS, 3 * OUT_C), jnp.float32),
        mesh=mesh,
        compiler_params=pltpu.CompilerParams(needs_layout_passes=False),
        scratch_types=[
            pltpu.VMEM((2, N), jnp.float32),
            pltpu.VMEM((320,), jnp.int32),
            pltpu.VMEM((2, NS), jnp.int32),
            pltpu.VMEM((2, NS, OUT_C), jnp.float32),
            pltpu.VMEM((2, 3 * OUT_C), jnp.float32),
            pltpu.SemaphoreType.DMA((2,)),
            pltpu.SemaphoreType.DMA((2,)),
            pltpu.SemaphoreType.DMA((2,)),
        ],
    )
    return f(d, p)


def _final_body(p_ref, q_ref, g_ref, b_ref, o_ref):
    s1 = p_ref[:, 0:OUT_C]
    s2 = p_ref[:, OUT_C:2 * OUT_C]
    mx = p_ref[:, 2 * OUT_C:3 * OUT_C]
    q = q_ref[...]
    cnt = jnp.float32(GROUPS * NS)
    sum_y = jnp.sum(s1, axis=0, keepdims=True) \
        - NS * jnp.sum(q, axis=0, keepdims=True)
    sum_y2 = jnp.sum(s2 - 2.0 * q * s1 + NS * q * q, axis=0, keepdims=True)
    mean = sum_y / cnt
    var = sum_y2 / cnt - mean * mean
    scale = g_ref[...] * lax.rsqrt(var + 1e-5)
    o_ref[...] = jnp.maximum((mx - q - mean) * scale + b_ref[...], 0.0)


def _finalize(packed, q, gamma, beta):
    return pl.pallas_call(
        _final_body,
        out_shape=jax.ShapeDtypeStruct((GROUPS, OUT_C), jnp.float32),
    )(packed, q, gamma.reshape(1, OUT_C), beta.reshape(1, OUT_C))


def kernel(xyz, t, points, W1, b1, convW, convb, gamma, beta):
    del t
    nx, ny, nz = _fps(xyz)
    new_xyz = jnp.stack([nx, ny, nz], axis=-1)
    feat = jnp.concatenate([xyz, points], axis=-1)
    p, q, d = _dense(feat, new_xyz, xyz, W1, convW, b1, convb)
    packed = _sc_groups(d.reshape(GROUPS, N), p.reshape(B * N, OUT_C))
    out = _finalize(packed, q.reshape(GROUPS, OUT_C), gamma, beta)
    return new_xyz, out.reshape(B, S, OUT_C)
